# baseline (speedup 1.0000x reference)
import jax
import jax.numpy as jnp
from jax.experimental import pallas as pl
from jax.experimental.pallas import tpu as pltpu

_NSEL = 64
_STRIDE = 1000
_T = 8
_TAIL_CHUNK = 1000


def _body(x_hbm, o_hbm, tiles_vmem, sem_bulk, sem_g, sem_s):
    rows, cols = x_hbm.shape
    gathers = []
    for k in range(_NSEL):
        g = pltpu.make_async_copy(
            x_hbm.at[pl.ds(k * _STRIDE, _T), :],
            tiles_vmem.at[pl.ds(k * _T, _T), :],
            sem_g,
        )
        g.start()
        gathers.append(g)
    bulk = []
    for k in range(_NSEL):
        c = pltpu.make_async_copy(
            x_hbm.at[pl.ds(k * _STRIDE + _T, _STRIDE - _T), :],
            o_hbm.at[pl.ds(k * _STRIDE + _T, _STRIDE - _T), :],
            sem_bulk,
        )
        c.start()
        bulk.append(c)
    tail_start = _NSEL * _STRIDE
    for s in range(tail_start, rows, _TAIL_CHUNK):
        ln = min(_TAIL_CHUNK, rows - s)
        c = pltpu.make_async_copy(
            x_hbm.at[pl.ds(s, ln), :],
            o_hbm.at[pl.ds(s, ln), :],
            sem_bulk,
        )
        c.start()
        bulk.append(c)
    for g in gathers:
        g.wait()
    tv = tiles_vmem[...]
    rid = jax.lax.broadcasted_iota(jnp.int32, tv.shape, 0)
    tiles_vmem[...] = jnp.where(rid % _T == 0, jnp.maximum(tv, 0.0), tv)
    scats = []
    for k in range(_NSEL):
        s = pltpu.make_async_copy(
            tiles_vmem.at[pl.ds(k * _T, _T), :],
            o_hbm.at[pl.ds(k * _STRIDE, _T), :],
            sem_s,
        )
        s.start()
        scats.append(s)
    for s in scats:
        s.wait()
    for c in bulk:
        c.wait()


def kernel(x):
    rows, cols = x.shape
    return pl.pallas_call(
        _body,
        in_specs=[pl.BlockSpec(memory_space=pltpu.HBM)],
        out_specs=pl.BlockSpec(memory_space=pltpu.HBM),
        out_shape=jax.ShapeDtypeStruct(x.shape, x.dtype),
        scratch_shapes=[
            pltpu.VMEM((_NSEL * _T, cols), jnp.float32),
            pltpu.SemaphoreType.DMA,
            pltpu.SemaphoreType.DMA,
            pltpu.SemaphoreType.DMA,
        ],
    )(x)


# speedup vs baseline: 44.0644x; 44.0644x over previous
import jax
import jax.numpy as jnp
from jax.experimental import pallas as pl

_BLOCK = 1000
_NSEL = 64
_T = 8


def _body(x_ref, o_ref):
    i = pl.program_id(0)
    o_ref[...] = x_ref[...]

    @pl.when(i < _NSEL)
    def _fix():
        slab = x_ref[0:_T, :]
        rid = jax.lax.broadcasted_iota(jnp.int32, slab.shape, 0)
        o_ref[0:_T, :] = jnp.where(rid == 0, jnp.maximum(slab, 0.0), slab)


def kernel(x):
    rows, cols = x.shape
    grid = rows // _BLOCK
    return pl.pallas_call(
        _body,
        grid=(grid,),
        in_specs=[pl.BlockSpec((_BLOCK, cols), lambda i: (i, 0))],
        out_specs=pl.BlockSpec((_BLOCK, cols), lambda i: (i, 0)),
        out_shape=jax.ShapeDtypeStruct(x.shape, x.dtype),
    )(x)


# speedup vs baseline: 48.9171x; 1.1101x over previous
import jax
import jax.numpy as jnp
from jax.experimental import pallas as pl
from jax.experimental.pallas import tpu as pltpu

_BLOCK = 5000
_PER = 5
_NSEL = 64
_T = 8


def _body(x_ref, o_ref):
    i = pl.program_id(0)
    o_ref[...] = x_ref[...]
    for j in range(_PER):
        @pl.when(i * _PER + j < _NSEL)
        def _fix(j=j):
            r = j * 1000
            slab = x_ref[r:r + _T, :]
            rid = jax.lax.broadcasted_iota(jnp.int32, slab.shape, 0)
            o_ref[r:r + _T, :] = jnp.where(rid == 0, jnp.maximum(slab, 0.0), slab)


def kernel(x):
    rows, cols = x.shape
    grid = rows // _BLOCK
    return pl.pallas_call(
        _body,
        grid=(grid,),
        in_specs=[pl.BlockSpec((_BLOCK, cols), lambda i: (i, 0))],
        out_specs=pl.BlockSpec((_BLOCK, cols), lambda i: (i, 0)),
        out_shape=jax.ShapeDtypeStruct(x.shape, x.dtype),
        compiler_params=pltpu.CompilerParams(vmem_limit_bytes=100 * 1024 * 1024),
    )(x)
